# Initial kernel scaffold; baseline (speedup 1.0000x reference)
import jax
import jax.numpy as jnp
from jax.experimental import pallas as pl


def kernel(hidden_states, weight, e_score_correction_bias):
    raise NotImplementedError("write your pallas kernel here")


# speedup vs baseline: 3.5475x; 3.5475x over previous
import jax
import jax.numpy as jnp
from jax.experimental import pallas as pl

_N_GROUP = 8
_GS = 8
_TOPK_GROUP = 4
_TOP_K = 8
_SCALE = 2.5
_E = 64


def _rot_within_group(v, s):
    r = jax.lax.broadcasted_iota(jnp.int32, v.shape, v.ndim - 1) % _GS
    w = jnp.roll(v, -s, axis=-1)
    u = jnp.roll(v, _GS - s, axis=-1)
    return jnp.where(r < _GS - s, w, u)


def _gate_kernel(h_ref, w_ref, b_ref, o_ref):
    h = h_ref[...]
    w = w_ref[...]
    logits = jax.lax.dot_general(
        h, w, (((1,), (1,)), ((), ())), preferred_element_type=jnp.float32)
    scores = jax.nn.sigmoid(logits)
    swb = scores + b_ref[...]
    bT = swb.shape[0]
    lane = jax.lax.broadcasted_iota(jnp.int32, (bT, _E), 1)
    g = lane // _GS

    m1 = swb
    m2 = jnp.full_like(swb, -jnp.inf)
    for s in (1, 2, 4):
        o1 = _rot_within_group(m1, s)
        o2 = _rot_within_group(m2, s)
        nm1 = jnp.maximum(m1, o1)
        nm2 = jnp.maximum(jnp.minimum(m1, o1), jnp.maximum(m2, o2))
        m1, m2 = nm1, nm2
    gs_full = m1 + m2

    grank = jnp.zeros_like(swb)
    for k in range(1, _N_GROUP):
        other = jnp.roll(gs_full, -_GS * k, axis=-1)
        og = (g + k) % _N_GROUP
        beats = (other > gs_full) | ((other == gs_full) & (og < g))
        grank = grank + jnp.where(beats, 1.0, 0.0)
    masked = jnp.where(grank < _TOPK_GROUP, swb, 0.0)

    remaining = masked
    selmask = jnp.zeros_like(swb, dtype=jnp.bool_)
    for _ in range(_TOP_K):
        m = jnp.max(remaining, axis=-1, keepdims=True)
        cand = remaining == m
        idx = jnp.min(jnp.where(cand, lane, _E), axis=-1, keepdims=True)
        pick = lane == idx
        selmask = selmask | pick
        remaining = jnp.where(pick, -jnp.inf, remaining)

    selected = jnp.where(selmask, scores, 0.0)
    ssum = jnp.sum(selected, axis=-1, keepdims=True) + 1e-20
    o_ref[...] = selected / ssum * _SCALE


def kernel(hidden_states, weight, e_score_correction_bias):
    T, H = hidden_states.shape
    E = weight.shape[0]
    bT = 256
    bias2 = e_score_correction_bias.reshape(1, E).astype(jnp.float32)
    return pl.pallas_call(
        _gate_kernel,
        grid=(T // bT,),
        in_specs=[
            pl.BlockSpec((bT, H), lambda i: (i, 0)),
            pl.BlockSpec((E, H), lambda i: (0, 0)),
            pl.BlockSpec((1, E), lambda i: (0, 0)),
        ],
        out_specs=pl.BlockSpec((bT, E), lambda i: (i, 0)),
        out_shape=jax.ShapeDtypeStruct((T, E), jnp.float32),
    )(hidden_states.astype(jnp.float32), weight.astype(jnp.float32), bias2)


# speedup vs baseline: 6.9675x; 1.9641x over previous
import jax
import jax.numpy as jnp
from jax.experimental import pallas as pl

_N_GROUP = 8
_GS = 8
_TOPK_GROUP = 4
_TOP_K = 8
_SCALE = 2.5
_E = 64


def _rot_rows_within_group(v, s):
    r = jax.lax.broadcasted_iota(jnp.int32, v.shape, 0) % _GS
    w = jnp.roll(v, -s, axis=0)
    u = jnp.roll(v, _GS - s, axis=0)
    return jnp.where(r < _GS - s, w, u)


def _gate_kernel(h_ref, w_ref, b_ref, o_ref):
    h = h_ref[...]
    w = w_ref[...]
    logits_t = jax.lax.dot_general(
        w, h, (((1,), (1,)), ((), ())), preferred_element_type=jnp.float32)
    scores = jax.nn.sigmoid(logits_t)
    swb = scores + b_ref[...]
    E, bT = swb.shape
    row = jax.lax.broadcasted_iota(jnp.int32, (E, bT), 0)
    g = row // _GS

    m1 = swb
    m2 = jnp.full_like(swb, -jnp.inf)
    for s in (1, 2, 4):
        o1 = _rot_rows_within_group(m1, s)
        o2 = _rot_rows_within_group(m2, s)
        nm1 = jnp.maximum(m1, o1)
        nm2 = jnp.maximum(jnp.minimum(m1, o1), jnp.maximum(m2, o2))
        m1, m2 = nm1, nm2
    gs_full = m1 + m2

    grank = jnp.zeros_like(swb)
    for k in range(1, _N_GROUP):
        other = jnp.roll(gs_full, -_GS * k, axis=0)
        og = (g + k) % _N_GROUP
        beats = (other > gs_full) | ((other == gs_full) & (og < g))
        grank = grank + jnp.where(beats, 1.0, 0.0)
    masked = jnp.where(grank < _TOPK_GROUP, swb, 0.0)

    remaining = masked
    selmask = jnp.zeros_like(swb, dtype=jnp.bool_)
    for _ in range(_TOP_K):
        m = jnp.max(remaining, axis=0, keepdims=True)
        cand = remaining == m
        idx = jnp.min(jnp.where(cand, row, E), axis=0, keepdims=True)
        pick = row == idx
        selmask = selmask | pick
        remaining = jnp.where(pick, -jnp.inf, remaining)

    selected = jnp.where(selmask, scores, 0.0)
    ssum = jnp.sum(selected, axis=0, keepdims=True) + 1e-20
    out = selected / ssum * _SCALE
    o_ref[...] = out.T


def kernel(hidden_states, weight, e_score_correction_bias):
    T, H = hidden_states.shape
    E = weight.shape[0]
    bT = 256
    bias2 = e_score_correction_bias.reshape(E, 1).astype(jnp.float32)
    return pl.pallas_call(
        _gate_kernel,
        grid=(T // bT,),
        in_specs=[
            pl.BlockSpec((bT, H), lambda i: (i, 0)),
            pl.BlockSpec((E, H), lambda i: (0, 0)),
            pl.BlockSpec((E, 1), lambda i: (0, 0)),
        ],
        out_specs=pl.BlockSpec((bT, E), lambda i: (i, 0)),
        out_shape=jax.ShapeDtypeStruct((T, E), jnp.float32),
    )(hidden_states.astype(jnp.float32), weight.astype(jnp.float32), bias2)


# speedup vs baseline: 8.2083x; 1.1781x over previous
import jax
import jax.numpy as jnp
from jax.experimental import pallas as pl

_N_GROUP = 8
_GS = 8
_TOPK_GROUP = 4
_TOP_K = 8
_SCALE = 2.5
_E = 64


def _rot_rows_within_group(v, s):
    r = jax.lax.broadcasted_iota(jnp.int32, v.shape, 0) % _GS
    w = jnp.roll(v, -s, axis=0)
    u = jnp.roll(v, _GS - s, axis=0)
    return jnp.where(r < _GS - s, w, u)


def _gate_kernel(h_ref, w_ref, b_ref, o_ref):
    h = h_ref[...]
    w = w_ref[...]
    logits_t = jax.lax.dot_general(
        w, h, (((1,), (1,)), ((), ())), preferred_element_type=jnp.float32)
    scores = jax.nn.sigmoid(logits_t)
    swb = scores + b_ref[...]
    E, bT = swb.shape
    row = jax.lax.broadcasted_iota(jnp.int32, (E, bT), 0)
    g = row // _GS

    m1 = swb
    m2 = jnp.full_like(swb, -jnp.inf)
    for s in (1, 2, 4):
        o1 = _rot_rows_within_group(m1, s)
        o2 = _rot_rows_within_group(m2, s)
        nm1 = jnp.maximum(m1, o1)
        nm2 = jnp.maximum(jnp.minimum(m1, o1), jnp.maximum(m2, o2))
        m1, m2 = nm1, nm2
    gs_full = m1 + m2

    grank = jnp.zeros_like(swb)
    for k in range(1, _N_GROUP):
        other = jnp.roll(gs_full, -_GS * k, axis=0)
        og = (g + k) % _N_GROUP
        beats = (other > gs_full) | ((other == gs_full) & (og < g))
        grank = grank + jnp.where(beats, 1.0, 0.0)
    masked = jnp.where(grank < _TOPK_GROUP, swb, 0.0)

    remaining = masked
    selmask = jnp.zeros_like(swb, dtype=jnp.bool_)
    for _ in range(_TOP_K):
        m = jnp.max(remaining, axis=0, keepdims=True)
        cand = remaining == m
        idx = jnp.min(jnp.where(cand, row, E), axis=0, keepdims=True)
        pick = row == idx
        selmask = selmask | pick
        remaining = jnp.where(pick, -jnp.inf, remaining)

    selected = jnp.where(selmask, scores, 0.0)
    ssum = jnp.sum(selected, axis=0, keepdims=True) + 1e-20
    out = selected / ssum * _SCALE
    o_ref[...] = out.T


def kernel(hidden_states, weight, e_score_correction_bias):
    T, H = hidden_states.shape
    E = weight.shape[0]
    bT = 512
    bias2 = e_score_correction_bias.reshape(E, 1).astype(jnp.float32)
    return pl.pallas_call(
        _gate_kernel,
        grid=(T // bT,),
        in_specs=[
            pl.BlockSpec((bT, H), lambda i: (i, 0)),
            pl.BlockSpec((E, H), lambda i: (0, 0)),
            pl.BlockSpec((E, 1), lambda i: (0, 0)),
        ],
        out_specs=pl.BlockSpec((bT, E), lambda i: (i, 0)),
        out_shape=jax.ShapeDtypeStruct((T, E), jnp.float32),
    )(hidden_states.astype(jnp.float32), weight.astype(jnp.float32), bias2)


# speedup vs baseline: 8.9178x; 1.0864x over previous
import jax
import jax.numpy as jnp
from jax.experimental import pallas as pl

_N_GROUP = 8
_GS = 8
_TOPK_GROUP = 4
_TOP_K = 8
_SCALE = 2.5
_E = 64


def _rot_rows_within_group(v, s):
    r = jax.lax.broadcasted_iota(jnp.int32, v.shape, 0) % _GS
    w = jnp.roll(v, -s, axis=0)
    u = jnp.roll(v, _GS - s, axis=0)
    return jnp.where(r < _GS - s, w, u)


def _gate_kernel(h_ref, w_ref, b_ref, o_ref):
    h = h_ref[...]
    w = w_ref[...]
    logits_t = jax.lax.dot_general(
        w, h, (((1,), (1,)), ((), ())), preferred_element_type=jnp.float32)
    scores = jax.nn.sigmoid(logits_t)
    swb = scores + b_ref[...]
    E, bT = swb.shape
    row = jax.lax.broadcasted_iota(jnp.int32, (E, bT), 0)
    g = row // _GS

    m1 = swb
    m2 = jnp.full_like(swb, -jnp.inf)
    for s in (1, 2, 4):
        o1 = _rot_rows_within_group(m1, s)
        o2 = _rot_rows_within_group(m2, s)
        nm1 = jnp.maximum(m1, o1)
        nm2 = jnp.maximum(jnp.minimum(m1, o1), jnp.maximum(m2, o2))
        m1, m2 = nm1, nm2
    gs_full = m1 + m2

    grank = jnp.zeros_like(swb)
    for k in range(1, _N_GROUP):
        other = jnp.roll(gs_full, -_GS * k, axis=0)
        og = (g + k) % _N_GROUP
        beats = (other > gs_full) | ((other == gs_full) & (og < g))
        grank = grank + jnp.where(beats, 1.0, 0.0)
    masked = jnp.where(grank < _TOPK_GROUP, swb, 0.0)

    remaining = masked
    selmask = jnp.zeros_like(swb, dtype=jnp.bool_)
    for _ in range(_TOP_K):
        m = jnp.max(remaining, axis=0, keepdims=True)
        cand = remaining == m
        idx = jnp.min(jnp.where(cand, row, E), axis=0, keepdims=True)
        pick = row == idx
        selmask = selmask | pick
        remaining = jnp.where(pick, -jnp.inf, remaining)

    selected = jnp.where(selmask, scores, 0.0)
    ssum = jnp.sum(selected, axis=0, keepdims=True) + 1e-20
    out = selected / ssum * _SCALE
    o_ref[...] = out.T


def kernel(hidden_states, weight, e_score_correction_bias):
    T, H = hidden_states.shape
    E = weight.shape[0]
    bT = 1024
    bias2 = e_score_correction_bias.reshape(E, 1).astype(jnp.float32)
    return pl.pallas_call(
        _gate_kernel,
        grid=(T // bT,),
        in_specs=[
            pl.BlockSpec((bT, H), lambda i: (i, 0)),
            pl.BlockSpec((E, H), lambda i: (0, 0)),
            pl.BlockSpec((E, 1), lambda i: (0, 0)),
        ],
        out_specs=pl.BlockSpec((bT, E), lambda i: (i, 0)),
        out_shape=jax.ShapeDtypeStruct((T, E), jnp.float32),
    )(hidden_states.astype(jnp.float32), weight.astype(jnp.float32), bias2)
